# baseline (speedup 1.0000x reference)
import functools

import jax
import jax.numpy as jnp
from jax import lax
from jax.experimental import pallas as pl
from jax.experimental.pallas import tpu as pltpu
from jax.experimental.pallas import tpu_sc as plsc

_V = 1000000
_D = 32
_B = 4096
_L = 200

_NC = 2
_NS = 16
_NW = _NC * _NS
_BPW = _B // _NW
_LANES = 16

_G = 8
_GTOK = _G * _L
_GPAD = 1664
_NGRP = _B // _G
_GPW = _BPW // _G


def _body(
    xp_hbm, lens_hbm, table_hbm, out_hbm, idx0, idx1, lens_v, buf0, buf1, out_v, sems
):
    wid = lax.axis_index("s") * _NC + lax.axis_index("c")
    base = wid * _BPW
    gbase = wid * _GPW

    pltpu.sync_copy(lens_hbm.at[pl.ds(base * _LANES, _BPW * _LANES)], lens_v)

    def stage_and_fire(g_local, idx_v, buf_v, sem):
        pltpu.sync_copy(xp_hbm.at[gbase + g_local], idx_v)
        pltpu.make_async_copy(table_hbm.at[idx_v], buf_v, sem).start()

    def drain(idx_v, buf_v, sem):
        pltpu.make_async_copy(table_hbm.at[idx_v], buf_v, sem).wait()

    def accumulate(g_local, buf_v):
        def row_body(r, _):
            row = g_local * _G + r

            def acc_body(i, carry):
                a0, a1 = carry
                p0 = r * _L + i * 8
                for jj in range(8):
                    p = p0 + jj
                    a0 = a0 + buf_v[p, pl.ds(0, _LANES)]
                    a1 = a1 + buf_v[p, pl.ds(_LANES, _LANES)]
                return (a0, a1)

            zero = jnp.zeros((_LANES,), jnp.float32)
            a0, a1 = lax.fori_loop(0, _L // 8, acc_body, (zero, zero))
            linv = 1.0 / lens_v[pl.ds(row * _LANES, _LANES)]
            out_v[pl.ds(row * _D, _LANES)] = a0 * linv
            out_v[pl.ds(row * _D + _LANES, _LANES)] = a1 * linv
            return 0

        lax.fori_loop(0, _G, row_body, 0)

    slots = ((idx0, buf0), (idx1, buf1))
    stage_and_fire(0, idx0, buf0, sems.at[0])
    stage_and_fire(1, idx1, buf1, sems.at[1])

    def pair_body(p, _):
        g0 = 2 * p
        for s in range(2):
            idx_v, buf_v = slots[s]
            drain(idx_v, buf_v, sems.at[s])
            accumulate(g0 + s, buf_v)

            @pl.when(g0 + s + 2 < _GPW)
            def _():
                stage_and_fire(g0 + s + 2, idx_v, buf_v, sems.at[s])

        return 0

    lax.fori_loop(0, _GPW // 2, pair_body, 0)

    pltpu.sync_copy(out_v, out_hbm.at[pl.ds(base * _D, _BPW * _D)])


_wordvec_avg = functools.partial(
    pl.kernel,
    out_type=jax.ShapeDtypeStruct((_B * _D,), jnp.float32),
    mesh=plsc.VectorSubcoreMesh(
        core_axis_name="c", subcore_axis_name="s", num_cores=_NC, num_subcores=_NS
    ),
    scratch_types=[
        pltpu.VMEM((_GPAD,), jnp.int32),
        pltpu.VMEM((_GPAD,), jnp.int32),
        pltpu.VMEM((_BPW * _LANES,), jnp.float32),
        pltpu.VMEM((_GPAD, _D), jnp.float32),
        pltpu.VMEM((_GPAD, _D), jnp.float32),
        pltpu.VMEM((_BPW * _D,), jnp.float32),
        pltpu.SemaphoreType.DMA((2,)),
    ],
    compiler_params=pltpu.CompilerParams(use_tc_tiling_on_sc=False),
)(_body)


def kernel(x, x_lens, table):
    xg = x.reshape(_NGRP, _GTOK)
    pad = jnp.zeros((_NGRP, _GPAD - _GTOK), jnp.int32)
    xp = jnp.concatenate([xg, pad], axis=1).reshape(_NGRP, _GPAD)
    lens_rep = jnp.repeat(x_lens, _LANES)
    return _wordvec_avg(xp, lens_rep, table).reshape(_B, _D)


# speedup vs baseline: 1.6035x; 1.6035x over previous
import functools

import jax
import jax.numpy as jnp
from jax import lax
from jax.experimental import pallas as pl
from jax.experimental.pallas import tpu as pltpu
from jax.experimental.pallas import tpu_sc as plsc

_V = 1000000
_D = 32
_B = 4096
_L = 200

_NC = 2
_NS = 16
_NW = _NC * _NS
_BPW = _B // _NW
_C0 = 128
_C1 = _L - _C0
_LANES = 16

_NSLOTS = 8


def _body(x_hbm, lens_hbm, table_hbm, out_hbm, xblk_v, lens_v, buf_v, out_v, sems):
    wid = lax.axis_index("s") * _NC + lax.axis_index("c")
    base = wid * _BPW

    pltpu.sync_copy(x_hbm.at[pl.ds(base, _BPW)], xblk_v)
    pltpu.sync_copy(lens_hbm.at[pl.ds(base * _LANES, _BPW * _LANES)], lens_v)

    def gather_row(b, slot):
        return (
            pltpu.make_async_copy(
                table_hbm.at[xblk_v.at[b, pl.ds(0, _C0)]],
                buf_v.at[slot, pl.ds(0, _C0)],
                sems.at[slot],
            ),
            pltpu.make_async_copy(
                table_hbm.at[xblk_v.at[b, pl.ds(_C0, _C1)]],
                buf_v.at[slot, pl.ds(_C0, _C1)],
                sems.at[slot],
            ),
        )

    def fire(b, slot):
        h0, h1 = gather_row(b, slot)
        h0.start()
        h1.start()

    def drain(b, slot):
        h0, h1 = gather_row(b, slot)
        h0.wait()
        h1.wait()

    def accumulate(b, slot):
        def acc_body(i, carry):
            a0, a1 = carry
            t0 = i * 8
            for j in range(8):
                a0 = a0 + buf_v[slot, t0 + j, pl.ds(0, _LANES)]
                a1 = a1 + buf_v[slot, t0 + j, pl.ds(_LANES, _LANES)]
            return (a0, a1)

        zero = jnp.zeros((_LANES,), jnp.float32)
        a0, a1 = lax.fori_loop(0, _L // 8, acc_body, (zero, zero))
        linv = 1.0 / lens_v[pl.ds(b * _LANES, _LANES)]
        out_v[pl.ds(b * _D, _LANES)] = a0 * linv
        out_v[pl.ds(b * _D + _LANES, _LANES)] = a1 * linv

    for s in range(_NSLOTS):
        fire(s, s)

    def group_body(g, _):
        r0 = g * _NSLOTS
        for s in range(_NSLOTS):
            drain(r0 + s, s)
            accumulate(r0 + s, s)

            @pl.when(r0 + s + _NSLOTS < _BPW)
            def _():
                fire(r0 + s + _NSLOTS, s)

        return 0

    lax.fori_loop(0, _BPW // _NSLOTS, group_body, 0)

    pltpu.sync_copy(out_v, out_hbm.at[pl.ds(base * _D, _BPW * _D)])


_wordvec_avg = functools.partial(
    pl.kernel,
    out_type=jax.ShapeDtypeStruct((_B * _D,), jnp.float32),
    mesh=plsc.VectorSubcoreMesh(
        core_axis_name="c", subcore_axis_name="s", num_cores=_NC, num_subcores=_NS
    ),
    scratch_types=[
        pltpu.VMEM((_BPW, _L), jnp.int32),
        pltpu.VMEM((_BPW * _LANES,), jnp.float32),
        pltpu.VMEM((_NSLOTS, _L, _D), jnp.float32),
        pltpu.VMEM((_BPW * _D,), jnp.float32),
        pltpu.SemaphoreType.DMA((_NSLOTS,)),
    ],
    compiler_params=pltpu.CompilerParams(use_tc_tiling_on_sc=False),
)(_body)


def kernel(x, x_lens, table):
    lens_rep = jnp.repeat(x_lens, _LANES)
    return _wordvec_avg(x, lens_rep, table).reshape(_B, _D)
